# baseline (speedup 1.0000x reference)
import functools

import jax
import jax.numpy as jnp
from jax import lax
from jax.experimental import pallas as pl
from jax.experimental.pallas import tpu as pltpu
from jax.experimental.pallas import tpu_sc as plsc

N_USERS = 20000
N_ITEMS = 30000
N_NODES = N_USERS + N_ITEMS
D = 32
N_EDGES = 1600000
N_LAY = 3

NC = 2
NS = 16
NW = NC * NS
EPT = N_EDGES // NW
CHUNK = 80
NCHUNK = EPT // CHUNK
STRIPE = N_NODES // NS
F = N_NODES * D
FPT = F // NW
SUB = 10000
NSUB = FPT // SUB

_MESH = plsc.VectorSubcoreMesh(core_axis_name="c", subcore_axis_name="s")


BPB = 5
BLKE = BPB * CHUNK
NBLK = EPT // BLKE
IRING = 4
RRING = 2


def _propagate_body(ego, src_r, dst2_r, vals_r, zeros_r,
                    partials, accum, sidx, didx, vb, rows,
                    sem_ld, sem_g, sem_sc):
    c = lax.axis_index("c")
    s = lax.axis_index("s")
    wid = s * NC + c

    pltpu.sync_copy(zeros_r, accum.at[pl.ds(s * STRIPE, STRIPE)])
    plsc.subcore_barrier()

    ebase = wid * EPT
    rbase = wid * (EPT // CHUNK)

    def issue_loads(blk):
        base = ebase + blk * BLKE
        slot = lax.rem(blk, IRING)
        pltpu.async_copy(src_r.at[pl.ds(base, BLKE)], sidx.at[slot], sem_ld)
        pltpu.async_copy(dst2_r.at[pl.ds(rbase + blk * BPB, BPB)],
                         didx.at[slot], sem_ld)
        pltpu.async_copy(vals_r.at[pl.ds(base, BLKE)],
                         vb.at[slot, pl.ds(16, BLKE)], sem_ld)

    def drain_loads():
        pltpu.make_async_copy(src_r.at[pl.ds(0, BLKE)], sidx.at[0], sem_ld).wait()
        pltpu.make_async_copy(dst2_r.at[pl.ds(0, BPB)], didx.at[0], sem_ld).wait()
        pltpu.make_async_copy(vals_r.at[pl.ds(0, BLKE)],
                              vb.at[0, pl.ds(16, BLKE)], sem_ld).wait()

    def issue_gathers(blk):
        islot = lax.rem(blk, IRING)
        rslot = lax.rem(blk, RRING)
        for c2 in range(BPB):
            pltpu.async_copy(ego.at[sidx.at[islot, pl.ds(c2 * CHUNK, CHUNK)]],
                             rows.at[rslot, c2], sem_g)

    def drain_rows(sem):
        for _ in range(BPB):
            pltpu.make_async_copy(ego.at[pl.ds(0, CHUNK)], rows.at[0, 0],
                                  sem).wait()

    issue_loads(0)
    issue_loads(1)
    drain_loads()
    issue_gathers(0)

    def block_body(blk, carry):
        @pl.when(blk >= 1)
        def _():
            drain_rows(sem_sc)

        @pl.when(blk < NBLK - 1)
        def _():
            drain_loads()
            issue_gathers(blk + 1)

        @pl.when(blk < NBLK - 2)
        def _():
            issue_loads(blk + 2)

        drain_rows(sem_g)
        islot = lax.rem(blk, IRING)
        rslot = lax.rem(blk, RRING)
        vbk = vb.at[islot]
        for c2 in range(BPB):
            for e in range(CHUNK):
                v = plsc.load_gather(
                    vbk, [jnp.full((16,), 16 + c2 * CHUNK + e, jnp.int32)])
                rows[rslot, c2, e, pl.ds(0, 16)] = (
                    rows[rslot, c2, e, pl.ds(0, 16)] * v)
                rows[rslot, c2, e, pl.ds(16, 16)] = (
                    rows[rslot, c2, e, pl.ds(16, 16)] * v)
        for c2 in range(BPB):
            pltpu.async_copy(rows.at[rslot, c2], accum.at[didx.at[islot, c2]],
                             sem_sc, add=True)
        return carry

    lax.fori_loop(0, NBLK, block_body, 0)
    drain_rows(sem_sc)
    plsc.subcore_barrier()
    pltpu.sync_copy(accum.at[pl.ds(s * STRIPE, STRIPE)],
                    partials.at[c, pl.ds(s * STRIPE, STRIPE)])


_propagate = functools.partial(
    pl.kernel,
    out_type=jax.ShapeDtypeStruct((NC, N_NODES, D), jnp.float32),
    mesh=_MESH,
    compiler_params=pltpu.CompilerParams(use_tc_tiling_on_sc=False, needs_layout_passes=False),
    scratch_types=[
        pltpu.VMEM_SHARED((N_NODES, D), jnp.float32),
        pltpu.VMEM((IRING, BLKE), jnp.int32),
        pltpu.VMEM((IRING, BPB, CHUNK), jnp.int32),
        pltpu.VMEM((IRING, BLKE + 16), jnp.float32),
        pltpu.VMEM((RRING, BPB, CHUNK, D), jnp.float32),
        pltpu.SemaphoreType.DMA,
        pltpu.SemaphoreType.DMA,
        pltpu.SemaphoreType.DMA,
    ],
)(_propagate_body)


def _merge_body(p_r, out_r, a, b):
    c = lax.axis_index("c")
    s = lax.axis_index("s")
    wid = s * NC + c

    def sub_body(j, carry):
        base = wid * FPT + j * SUB
        pltpu.sync_copy(p_r.at[0, pl.ds(base, SUB)], a)
        pltpu.sync_copy(p_r.at[1, pl.ds(base, SUB)], b)

        def add_body(k, carry2):
            o = k * 16
            a[pl.ds(o, 16)] = a[pl.ds(o, 16)] + b[pl.ds(o, 16)]
            return carry2

        lax.fori_loop(0, SUB // 16, add_body, 0)
        pltpu.sync_copy(a, out_r.at[pl.ds(base, SUB)])
        return carry

    lax.fori_loop(0, NSUB, sub_body, 0)


_merge = functools.partial(
    pl.kernel,
    out_type=jax.ShapeDtypeStruct((F,), jnp.float32),
    mesh=_MESH,
    compiler_params=pltpu.CompilerParams(use_tc_tiling_on_sc=False, needs_layout_passes=False),
    scratch_types=[
        pltpu.VMEM((SUB,), jnp.float32),
        pltpu.VMEM((SUB,), jnp.float32),
    ],
)(_merge_body)


def _merge_final_body(p_r, e1_r, e2_r, out3_r, mean_r, a, b, m):
    c = lax.axis_index("c")
    s = lax.axis_index("s")
    wid = s * NC + c
    third = jnp.float32(1.0 / 3.0)

    def sub_body(j, carry):
        base = wid * FPT + j * SUB
        pltpu.sync_copy(p_r.at[0, pl.ds(base, SUB)], a)
        pltpu.sync_copy(p_r.at[1, pl.ds(base, SUB)], b)

        def add_body(k, carry2):
            o = k * 16
            a[pl.ds(o, 16)] = a[pl.ds(o, 16)] + b[pl.ds(o, 16)]
            return carry2

        lax.fori_loop(0, SUB // 16, add_body, 0)
        pltpu.sync_copy(a, out3_r.at[pl.ds(base, SUB)])
        pltpu.sync_copy(e1_r.at[pl.ds(base, SUB)], b)
        pltpu.sync_copy(e2_r.at[pl.ds(base, SUB)], m)

        def mean_body(k, carry2):
            o = k * 16
            b[pl.ds(o, 16)] = (a[pl.ds(o, 16)] + b[pl.ds(o, 16)]
                               + m[pl.ds(o, 16)]) * third
            return carry2

        lax.fori_loop(0, SUB // 16, mean_body, 0)
        pltpu.sync_copy(b, mean_r.at[pl.ds(base, SUB)])
        return carry

    lax.fori_loop(0, NSUB, sub_body, 0)


_merge_final = functools.partial(
    pl.kernel,
    out_type=(jax.ShapeDtypeStruct((F,), jnp.float32),
              jax.ShapeDtypeStruct((F,), jnp.float32)),
    mesh=_MESH,
    compiler_params=pltpu.CompilerParams(use_tc_tiling_on_sc=False, needs_layout_passes=False),
    scratch_types=[
        pltpu.VMEM((SUB,), jnp.float32),
        pltpu.VMEM((SUB,), jnp.float32),
        pltpu.VMEM((SUB,), jnp.float32),
    ],
)(_merge_final_body)


def kernel(user_emb, item_emb, adj_vals, edge_index):
    ego = jnp.concatenate([user_emb, item_emb], axis=0)
    src = edge_index[1]
    dst2 = edge_index[0].reshape(N_EDGES // CHUNK, CHUNK)
    zeros = jnp.zeros((STRIPE, D), jnp.float32)

    layer_flat = []
    mean_flat = None
    for layer in range(N_LAY):
        p = _propagate(ego, src, dst2, adj_vals, zeros)
        pf = p.reshape(NC, F)
        if layer < N_LAY - 1:
            ef = _merge(pf)
            layer_flat.append(ef)
            ego = ef.reshape(N_NODES, D)
        else:
            e3f, mean_flat = _merge_final(pf, layer_flat[0], layer_flat[1])
            layer_flat.append(e3f)

    stacked = jnp.stack([f.reshape(N_NODES, D) for f in layer_flat], axis=1)
    all_e = mean_flat.reshape(N_NODES, D)
    return (all_e[:N_USERS], all_e[N_USERS:],
            stacked[:N_USERS], stacked[N_USERS:])


# speedup vs baseline: 2.5977x; 2.5977x over previous
import functools

import jax
import jax.numpy as jnp
from jax import lax
from jax.experimental import pallas as pl
from jax.experimental.pallas import tpu as pltpu
from jax.experimental.pallas import tpu_sc as plsc

N_USERS = 20000
N_ITEMS = 30000
N_NODES = N_USERS + N_ITEMS
D = 32
N_EDGES = 1600000
N_LAY = 3

NC = 2
NS = 16
NW = NC * NS
EPT = N_EDGES // NW
CHUNK = 80
NCHUNK = EPT // CHUNK
STRIPE = N_NODES // NS
F = N_NODES * D
FPT = F // NW
SUB = 10000
NSUB = FPT // SUB

_MESH = plsc.VectorSubcoreMesh(core_axis_name="c", subcore_axis_name="s")


BPB = 5
BLKE = BPB * CHUNK
NBLK = EPT // BLKE
BRING = 4
DBRING = 4
RRING = 6
SCD = 3


def _propagate_body(ego, src2_r, dst2_r, vals_r, zeros_r,
                    partials, accum, sidx, didx, vb, rows,
                    sem_ld, sem_g, sem_sc):
    c = lax.axis_index("c")
    s = lax.axis_index("s")
    wid = s * NC + c

    pltpu.sync_copy(zeros_r, accum.at[pl.ds(s * STRIPE, STRIPE)])
    plsc.subcore_barrier()

    ebase = wid * EPT
    rbase = wid * (EPT // CHUNK)

    def issue_loads(blk):
        rowb = rbase + blk * BPB
        pltpu.async_copy(src2_r.at[pl.ds(rowb, BPB)],
                         sidx.at[pl.ds(lax.rem(blk, BRING) * BPB, BPB)],
                         sem_ld)
        pltpu.async_copy(dst2_r.at[pl.ds(rowb, BPB)],
                         didx.at[pl.ds(lax.rem(blk, DBRING) * BPB, BPB)],
                         sem_ld)
        pltpu.async_copy(vals_r.at[pl.ds(ebase + blk * BLKE, BLKE)],
                         vb.at[pl.ds(lax.rem(blk, BRING) * BLKE + 16, BLKE)],
                         sem_ld)

    def drain_loads():
        pltpu.make_async_copy(src2_r.at[pl.ds(0, BPB)],
                              sidx.at[pl.ds(0, BPB)], sem_ld).wait()
        pltpu.make_async_copy(dst2_r.at[pl.ds(0, BPB)],
                              didx.at[pl.ds(0, BPB)], sem_ld).wait()
        pltpu.make_async_copy(vals_r.at[pl.ds(0, BLKE)],
                              vb.at[pl.ds(16, BLKE)], sem_ld).wait()

    def issue_gather(k):
        blk = lax.div(k, BPB)
        row = lax.rem(blk, BRING) * BPB + lax.rem(k, BPB)
        pltpu.async_copy(ego.at[sidx.at[row]],
                         rows.at[lax.rem(k, RRING)], sem_g)

    def drain_rows(sem):
        pltpu.make_async_copy(ego.at[pl.ds(0, CHUNK)], rows.at[0], sem).wait()

    issue_loads(0)
    issue_loads(1)
    drain_loads()
    issue_gather(0)
    issue_gather(1)

    def chunk_body(k, carry):
        @pl.when(k >= SCD)
        def _():
            drain_rows(sem_sc)

        @pl.when((lax.rem(k, BPB) == 0) & (k < (NBLK - 2) * BPB))
        def _():
            issue_loads(lax.div(k, BPB) + 2)

        @pl.when((lax.rem(k, BPB) == BPB - 2) & (k < NCHUNK - 3))
        def _():
            drain_loads()

        @pl.when(k < NCHUNK - 2)
        def _():
            issue_gather(k + 2)

        drain_rows(sem_g)
        blk = lax.div(k, BPB)
        boff = lax.rem(blk, BRING) * BLKE + lax.rem(k, BPB) * CHUNK + 16
        rk = lax.rem(k, RRING)
        for g in range(CHUNK // 16):
            vv = vb[pl.ds(boff + g * 16, 16)]
            for j in range(16):
                e = g * 16 + j
                v = lax.gather(
                    vv, jnp.full((16, 1), j, jnp.int32),
                    lax.GatherDimensionNumbers(
                        offset_dims=(), collapsed_slice_dims=(0,),
                        start_index_map=(0,)),
                    (1,), mode=lax.GatherScatterMode.PROMISE_IN_BOUNDS)
                rows[rk, e, pl.ds(0, 16)] = rows[rk, e, pl.ds(0, 16)] * v
                rows[rk, e, pl.ds(16, 16)] = rows[rk, e, pl.ds(16, 16)] * v
        drow = lax.rem(blk, DBRING) * BPB + lax.rem(k, BPB)
        pltpu.async_copy(rows.at[rk], accum.at[didx.at[drow]],
                         sem_sc, add=True)
        return carry

    lax.fori_loop(0, NCHUNK, chunk_body, 0)
    for _ in range(SCD):
        drain_rows(sem_sc)
    plsc.subcore_barrier()
    pltpu.sync_copy(accum.at[pl.ds(s * STRIPE, STRIPE)],
                    partials.at[c, pl.ds(s * STRIPE, STRIPE)])


_propagate = functools.partial(
    pl.kernel,
    out_type=jax.ShapeDtypeStruct((NC, N_NODES, D), jnp.float32),
    mesh=_MESH,
    compiler_params=pltpu.CompilerParams(use_tc_tiling_on_sc=False, needs_layout_passes=False),
    scratch_types=[
        pltpu.VMEM_SHARED((N_NODES, D), jnp.float32),
        pltpu.VMEM((BRING * BPB, CHUNK), jnp.int32),
        pltpu.VMEM((DBRING * BPB, CHUNK), jnp.int32),
        pltpu.VMEM((BRING * BLKE + 16,), jnp.float32),
        pltpu.VMEM((RRING, CHUNK, D), jnp.float32),
        pltpu.SemaphoreType.DMA,
        pltpu.SemaphoreType.DMA,
        pltpu.SemaphoreType.DMA,
    ],
)(_propagate_body)


def _merge_body(p_r, out_r, a, b):
    c = lax.axis_index("c")
    s = lax.axis_index("s")
    wid = s * NC + c

    def sub_body(j, carry):
        base = wid * FPT + j * SUB
        pltpu.sync_copy(p_r.at[0, pl.ds(base, SUB)], a)
        pltpu.sync_copy(p_r.at[1, pl.ds(base, SUB)], b)

        def add_body(k, carry2):
            o = k * 16
            a[pl.ds(o, 16)] = a[pl.ds(o, 16)] + b[pl.ds(o, 16)]
            return carry2

        lax.fori_loop(0, SUB // 16, add_body, 0)
        pltpu.sync_copy(a, out_r.at[pl.ds(base, SUB)])
        return carry

    lax.fori_loop(0, NSUB, sub_body, 0)


_merge = functools.partial(
    pl.kernel,
    out_type=jax.ShapeDtypeStruct((F,), jnp.float32),
    mesh=_MESH,
    compiler_params=pltpu.CompilerParams(use_tc_tiling_on_sc=False, needs_layout_passes=False),
    scratch_types=[
        pltpu.VMEM((SUB,), jnp.float32),
        pltpu.VMEM((SUB,), jnp.float32),
    ],
)(_merge_body)


def _merge_final_body(p_r, e1_r, e2_r, out3_r, mean_r, a, b, m):
    c = lax.axis_index("c")
    s = lax.axis_index("s")
    wid = s * NC + c
    third = jnp.float32(1.0 / 3.0)

    def sub_body(j, carry):
        base = wid * FPT + j * SUB
        pltpu.sync_copy(p_r.at[0, pl.ds(base, SUB)], a)
        pltpu.sync_copy(p_r.at[1, pl.ds(base, SUB)], b)

        def add_body(k, carry2):
            o = k * 16
            a[pl.ds(o, 16)] = a[pl.ds(o, 16)] + b[pl.ds(o, 16)]
            return carry2

        lax.fori_loop(0, SUB // 16, add_body, 0)
        pltpu.sync_copy(a, out3_r.at[pl.ds(base, SUB)])
        pltpu.sync_copy(e1_r.at[pl.ds(base, SUB)], b)
        pltpu.sync_copy(e2_r.at[pl.ds(base, SUB)], m)

        def mean_body(k, carry2):
            o = k * 16
            b[pl.ds(o, 16)] = (a[pl.ds(o, 16)] + b[pl.ds(o, 16)]
                               + m[pl.ds(o, 16)]) * third
            return carry2

        lax.fori_loop(0, SUB // 16, mean_body, 0)
        pltpu.sync_copy(b, mean_r.at[pl.ds(base, SUB)])
        return carry

    lax.fori_loop(0, NSUB, sub_body, 0)


_merge_final = functools.partial(
    pl.kernel,
    out_type=(jax.ShapeDtypeStruct((F,), jnp.float32),
              jax.ShapeDtypeStruct((F,), jnp.float32)),
    mesh=_MESH,
    compiler_params=pltpu.CompilerParams(use_tc_tiling_on_sc=False, needs_layout_passes=False),
    scratch_types=[
        pltpu.VMEM((SUB,), jnp.float32),
        pltpu.VMEM((SUB,), jnp.float32),
        pltpu.VMEM((SUB,), jnp.float32),
    ],
)(_merge_final_body)


def kernel(user_emb, item_emb, adj_vals, edge_index):
    ego = jnp.concatenate([user_emb, item_emb], axis=0)
    src2 = edge_index[1].reshape(N_EDGES // CHUNK, CHUNK)
    dst2 = edge_index[0].reshape(N_EDGES // CHUNK, CHUNK)
    zeros = jnp.zeros((STRIPE, D), jnp.float32)

    layer_flat = []
    mean_flat = None
    for layer in range(N_LAY):
        p = _propagate(ego, src2, dst2, adj_vals, zeros)
        pf = p.reshape(NC, F)
        if layer < N_LAY - 1:
            ef = _merge(pf)
            layer_flat.append(ef)
            ego = ef.reshape(N_NODES, D)
        else:
            e3f, mean_flat = _merge_final(pf, layer_flat[0], layer_flat[1])
            layer_flat.append(e3f)

    stacked = jnp.stack([f.reshape(N_NODES, D) for f in layer_flat], axis=1)
    all_e = mean_flat.reshape(N_NODES, D)
    return (all_e[:N_USERS], all_e[N_USERS:],
            stacked[:N_USERS], stacked[N_USERS:])


# speedup vs baseline: 2.6027x; 1.0019x over previous
import functools

import jax
import jax.numpy as jnp
from jax import lax
from jax.experimental import pallas as pl
from jax.experimental.pallas import tpu as pltpu
from jax.experimental.pallas import tpu_sc as plsc

N_USERS = 20000
N_ITEMS = 30000
N_NODES = N_USERS + N_ITEMS
D = 32
N_EDGES = 1600000
N_LAY = 3

NC = 2
NS = 16
NW = NC * NS
EPT = N_EDGES // NW
CHUNK = 80
NCHUNK = EPT // CHUNK
STRIPE = N_NODES // NS
F = N_NODES * D
FPT = F // NW
SUB = 10000
NSUB = FPT // SUB

_MESH = plsc.VectorSubcoreMesh(core_axis_name="c", subcore_axis_name="s")


BPB = 5
BLKE = BPB * CHUNK
NBLK = EPT // BLKE
BRING = 4
DBRING = 4
RRING = 6
SCD = 3


def _propagate_body(ego, src2_r, dst2_r, vals_r, zeros_r,
                    partials, accum, sidx, didx, vb, rows,
                    sem_ld, sem_g, sem_sc):
    c = lax.axis_index("c")
    s = lax.axis_index("s")
    wid = s * NC + c

    pltpu.sync_copy(zeros_r, accum.at[pl.ds(s * STRIPE, STRIPE)])
    plsc.subcore_barrier()

    ebase = wid * EPT
    rbase = wid * (EPT // CHUNK)

    def issue_loads(blk):
        rowb = rbase + blk * BPB
        pltpu.async_copy(src2_r.at[pl.ds(rowb, BPB)],
                         sidx.at[pl.ds(lax.rem(blk, BRING) * BPB, BPB)],
                         sem_ld)
        pltpu.async_copy(dst2_r.at[pl.ds(rowb, BPB)],
                         didx.at[pl.ds(lax.rem(blk, DBRING) * BPB, BPB)],
                         sem_ld)
        pltpu.async_copy(vals_r.at[pl.ds(ebase + blk * BLKE, BLKE)],
                         vb.at[pl.ds(lax.rem(blk, BRING) * BLKE + 16, BLKE)],
                         sem_ld)

    def drain_loads():
        pltpu.make_async_copy(src2_r.at[pl.ds(0, BPB)],
                              sidx.at[pl.ds(0, BPB)], sem_ld).wait()
        pltpu.make_async_copy(dst2_r.at[pl.ds(0, BPB)],
                              didx.at[pl.ds(0, BPB)], sem_ld).wait()
        pltpu.make_async_copy(vals_r.at[pl.ds(0, BLKE)],
                              vb.at[pl.ds(16, BLKE)], sem_ld).wait()

    def issue_gather(k):
        blk = lax.div(k, BPB)
        row = lax.rem(blk, BRING) * BPB + lax.rem(k, BPB)
        pltpu.async_copy(ego.at[sidx.at[row]],
                         rows.at[lax.rem(k, RRING)], sem_g)

    def drain_rows(sem):
        pltpu.make_async_copy(ego.at[pl.ds(0, CHUNK)], rows.at[0], sem).wait()

    issue_loads(0)
    issue_loads(1)
    drain_loads()
    issue_gather(0)
    issue_gather(1)

    def chunk_body(k, carry):
        @pl.when(k >= SCD)
        def _():
            drain_rows(sem_sc)

        @pl.when((lax.rem(k, BPB) == 0) & (k < (NBLK - 2) * BPB))
        def _():
            issue_loads(lax.div(k, BPB) + 2)

        @pl.when((lax.rem(k, BPB) == BPB - 2) & (k < NCHUNK - 3))
        def _():
            drain_loads()

        @pl.when(k < NCHUNK - 2)
        def _():
            issue_gather(k + 2)

        drain_rows(sem_g)
        blk = lax.div(k, BPB)
        boff = lax.rem(blk, BRING) * BLKE + lax.rem(k, BPB) * CHUNK + 16
        rk = lax.rem(k, RRING)
        for g in range(CHUNK // 16):
            vv = vb[pl.ds(boff + g * 16, 16)]
            for j in range(16):
                e = g * 16 + j
                v = lax.gather(
                    vv, jnp.full((16, 1), j, jnp.int32),
                    lax.GatherDimensionNumbers(
                        offset_dims=(), collapsed_slice_dims=(0,),
                        start_index_map=(0,)),
                    (1,), mode=lax.GatherScatterMode.PROMISE_IN_BOUNDS)
                rows[rk, e, pl.ds(0, 16)] = rows[rk, e, pl.ds(0, 16)] * v
                rows[rk, e, pl.ds(16, 16)] = rows[rk, e, pl.ds(16, 16)] * v
        drow = lax.rem(blk, DBRING) * BPB + lax.rem(k, BPB)
        pltpu.async_copy(rows.at[rk], accum.at[didx.at[drow]],
                         sem_sc, add=True)
        return carry

    lax.fori_loop(0, NCHUNK, chunk_body, 0)
    for _ in range(SCD):
        drain_rows(sem_sc)
    plsc.subcore_barrier()
    pltpu.sync_copy(accum.at[pl.ds(s * STRIPE, STRIPE)],
                    partials.at[c, pl.ds(s * STRIPE, STRIPE)])


_propagate = functools.partial(
    pl.kernel,
    out_type=jax.ShapeDtypeStruct((NC, N_NODES, D), jnp.float32),
    mesh=_MESH,
    compiler_params=pltpu.CompilerParams(use_tc_tiling_on_sc=False, needs_layout_passes=False),
    scratch_types=[
        pltpu.VMEM_SHARED((N_NODES, D), jnp.float32),
        pltpu.VMEM((BRING * BPB, CHUNK), jnp.int32),
        pltpu.VMEM((DBRING * BPB, CHUNK), jnp.int32),
        pltpu.VMEM((BRING * BLKE + 16,), jnp.float32),
        pltpu.VMEM((RRING, CHUNK, D), jnp.float32),
        pltpu.SemaphoreType.DMA,
        pltpu.SemaphoreType.DMA,
        pltpu.SemaphoreType.DMA,
    ],
)(_propagate_body)


MROWS = 1563
MSUB = 521
U_RPT = N_USERS // NW
I_SUB_A = 469
I_SUB_B = 468


def _merge_body(p_r, out_r, a, b):
    c = lax.axis_index("c")
    s = lax.axis_index("s")
    wid = s * NC + c

    def seg(r0, n):
        pltpu.sync_copy(p_r.at[0, pl.ds(r0, n)], a.at[pl.ds(0, n)])
        pltpu.sync_copy(p_r.at[1, pl.ds(r0, n)], b.at[pl.ds(0, n)])

        def add(r, carry):
            a[r, pl.ds(0, 16)] = a[r, pl.ds(0, 16)] + b[r, pl.ds(0, 16)]
            a[r, pl.ds(16, 16)] = a[r, pl.ds(16, 16)] + b[r, pl.ds(16, 16)]
            return carry

        lax.fori_loop(0, n, add, 0)
        pltpu.sync_copy(a.at[pl.ds(0, n)], out_r.at[pl.ds(r0, n)])

    @pl.when(wid < NW - 1)
    def _():
        for t in range(3):
            seg(wid * MROWS + t * MSUB, MSUB)

    @pl.when(wid == NW - 1)
    def _():
        base = (NW - 1) * MROWS
        seg(base, MSUB)
        seg(base + MSUB, MSUB)
        seg(base + 2 * MSUB, N_NODES - base - 2 * MSUB)


_merge = functools.partial(
    pl.kernel,
    out_type=jax.ShapeDtypeStruct((N_NODES, D), jnp.float32),
    mesh=_MESH,
    compiler_params=pltpu.CompilerParams(use_tc_tiling_on_sc=False, needs_layout_passes=False),
    scratch_types=[
        pltpu.VMEM((MSUB, D), jnp.float32),
        pltpu.VMEM((MSUB, D), jnp.float32),
    ],
)(_merge_body)


def _final_body(p_r, e1_r, e2_r, ua_r, ia_r, ul_r, il_r, b1, b2, b3, b4):
    c = lax.axis_index("c")
    s = lax.axis_index("s")
    wid = s * NC + c
    third = jnp.float32(1.0 / 3.0)

    def seg(in_r0, out_r0, n, lay_out, all_out):
        pltpu.sync_copy(e1_r.at[pl.ds(in_r0, n)], b1.at[pl.ds(0, n)])
        pltpu.sync_copy(e2_r.at[pl.ds(in_r0, n)], b2.at[pl.ds(0, n)])
        pltpu.sync_copy(p_r.at[0, pl.ds(in_r0, n)], b3.at[pl.ds(0, n)])
        pltpu.sync_copy(p_r.at[1, pl.ds(in_r0, n)], b4.at[pl.ds(0, n)])

        def add(r, carry):
            b3[r, pl.ds(0, 16)] = b3[r, pl.ds(0, 16)] + b4[r, pl.ds(0, 16)]
            b3[r, pl.ds(16, 16)] = b3[r, pl.ds(16, 16)] + b4[r, pl.ds(16, 16)]
            return carry

        lax.fori_loop(0, n, add, 0)

        def mean(r, carry):
            for h in (0, 16):
                b4[r, pl.ds(h, 16)] = (b1[r, pl.ds(h, 16)]
                                       + b2[r, pl.ds(h, 16)]
                                       + b3[r, pl.ds(h, 16)]) * third
            return carry

        lax.fori_loop(0, n, mean, 0)
        pltpu.sync_copy(b1.at[pl.ds(0, n)], lay_out.at[pl.ds(out_r0, n), 0])
        pltpu.sync_copy(b2.at[pl.ds(0, n)], lay_out.at[pl.ds(out_r0, n), 1])
        pltpu.sync_copy(b3.at[pl.ds(0, n)], lay_out.at[pl.ds(out_r0, n), 2])
        pltpu.sync_copy(b4.at[pl.ds(0, n)], all_out.at[pl.ds(out_r0, n)])

    seg(wid * U_RPT, wid * U_RPT, U_RPT, ul_r, ua_r)

    @pl.when(wid < NS)
    def _():
        r0 = wid * (2 * I_SUB_A)
        seg(N_USERS + r0, r0, I_SUB_A, il_r, ia_r)
        seg(N_USERS + r0 + I_SUB_A, r0 + I_SUB_A, I_SUB_A, il_r, ia_r)

    @pl.when(wid >= NS)
    def _():
        r0 = NS * (2 * I_SUB_A) + (wid - NS) * (I_SUB_A + I_SUB_B)
        seg(N_USERS + r0, r0, I_SUB_A, il_r, ia_r)
        seg(N_USERS + r0 + I_SUB_A, r0 + I_SUB_A, I_SUB_B, il_r, ia_r)


_final = functools.partial(
    pl.kernel,
    out_type=(jax.ShapeDtypeStruct((N_USERS, D), jnp.float32),
              jax.ShapeDtypeStruct((N_ITEMS, D), jnp.float32),
              jax.ShapeDtypeStruct((N_USERS, N_LAY, D), jnp.float32),
              jax.ShapeDtypeStruct((N_ITEMS, N_LAY, D), jnp.float32)),
    mesh=_MESH,
    compiler_params=pltpu.CompilerParams(use_tc_tiling_on_sc=False, needs_layout_passes=False),
    scratch_types=[
        pltpu.VMEM((U_RPT, D), jnp.float32),
        pltpu.VMEM((U_RPT, D), jnp.float32),
        pltpu.VMEM((U_RPT, D), jnp.float32),
        pltpu.VMEM((U_RPT, D), jnp.float32),
    ],
)(_final_body)


def kernel(user_emb, item_emb, adj_vals, edge_index):
    ego = jnp.concatenate([user_emb, item_emb], axis=0)
    src2 = edge_index[1].reshape(N_EDGES // CHUNK, CHUNK)
    dst2 = edge_index[0].reshape(N_EDGES // CHUNK, CHUNK)
    zeros = jnp.zeros((STRIPE, D), jnp.float32)

    p1 = _propagate(ego, src2, dst2, adj_vals, zeros)
    e1 = _merge(p1)
    p2 = _propagate(e1, src2, dst2, adj_vals, zeros)
    e2 = _merge(p2)
    p3 = _propagate(e2, src2, dst2, adj_vals, zeros)
    return _final(p3, e1, e2)


# speedup vs baseline: 2.7226x; 1.0461x over previous
import functools

import jax
import jax.numpy as jnp
from jax import lax
from jax.experimental import pallas as pl
from jax.experimental.pallas import tpu as pltpu
from jax.experimental.pallas import tpu_sc as plsc

N_USERS = 20000
N_ITEMS = 30000
N_NODES = N_USERS + N_ITEMS
D = 32
N_EDGES = 1600000
N_LAY = 3

NC = 2
NS = 16
NW = NC * NS
EPT = N_EDGES // NW
CHUNK = 80
NCHUNK = EPT // CHUNK
STRIPE = N_NODES // NS
F = N_NODES * D
FPT = F // NW
SUB = 10000
NSUB = FPT // SUB

_MESH = plsc.VectorSubcoreMesh(core_axis_name="c", subcore_axis_name="s")


BPB = 5
BLKE = BPB * CHUNK
NBLK = EPT // BLKE
BRING = 4
DBRING = 4
RRING = 6
SCD = 3


def _propagate_body(ego, ei_r, vals_r, zeros_r,
                    partials, accum, sidx, didx, vb, rows,
                    sem_ld, sem_g, sem_sc):
    c = lax.axis_index("c")
    s = lax.axis_index("s")
    wid = s * NC + c

    pltpu.sync_copy(zeros_r, accum.at[pl.ds(s * STRIPE, STRIPE)])
    plsc.subcore_barrier()

    ebase = wid * EPT

    def issue_loads(blk):
        base = ebase + blk * BLKE
        pltpu.async_copy(ei_r.at[1, pl.ds(base, BLKE)],
                         sidx.at[pl.ds(lax.rem(blk, BRING) * BLKE, BLKE)],
                         sem_ld)
        pltpu.async_copy(ei_r.at[0, pl.ds(base, BLKE)],
                         didx.at[pl.ds(lax.rem(blk, DBRING) * BLKE, BLKE)],
                         sem_ld)
        pltpu.async_copy(vals_r.at[pl.ds(base, BLKE)],
                         vb.at[pl.ds(lax.rem(blk, BRING) * BLKE + 16, BLKE)],
                         sem_ld)

    def drain_loads():
        pltpu.make_async_copy(ei_r.at[1, pl.ds(0, BLKE)],
                              sidx.at[pl.ds(0, BLKE)], sem_ld).wait()
        pltpu.make_async_copy(ei_r.at[0, pl.ds(0, BLKE)],
                              didx.at[pl.ds(0, BLKE)], sem_ld).wait()
        pltpu.make_async_copy(vals_r.at[pl.ds(0, BLKE)],
                              vb.at[pl.ds(16, BLKE)], sem_ld).wait()

    def issue_gather(k):
        blk = lax.div(k, BPB)
        off = lax.rem(blk, BRING) * BLKE + lax.rem(k, BPB) * CHUNK
        pltpu.async_copy(ego.at[sidx.at[pl.ds(off, CHUNK)]],
                         rows.at[lax.rem(k, RRING)], sem_g)

    def drain_rows(sem):
        pltpu.make_async_copy(ego.at[pl.ds(0, CHUNK)], rows.at[0], sem).wait()

    issue_loads(0)
    issue_loads(1)
    drain_loads()
    issue_gather(0)
    issue_gather(1)

    def chunk_body(k, carry):
        @pl.when(k >= SCD)
        def _():
            drain_rows(sem_sc)

        @pl.when((lax.rem(k, BPB) == 0) & (k < (NBLK - 2) * BPB))
        def _():
            issue_loads(lax.div(k, BPB) + 2)

        @pl.when((lax.rem(k, BPB) == BPB - 2) & (k < NCHUNK - 3))
        def _():
            drain_loads()

        @pl.when(k < NCHUNK - 2)
        def _():
            issue_gather(k + 2)

        drain_rows(sem_g)
        blk = lax.div(k, BPB)
        boff = lax.rem(blk, BRING) * BLKE + lax.rem(k, BPB) * CHUNK + 16
        rk = lax.rem(k, RRING)
        for g in range(CHUNK // 16):
            vv = vb[pl.ds(boff + g * 16, 16)]
            for j in range(16):
                e = g * 16 + j
                v = lax.gather(
                    vv, jnp.full((16, 1), j, jnp.int32),
                    lax.GatherDimensionNumbers(
                        offset_dims=(), collapsed_slice_dims=(0,),
                        start_index_map=(0,)),
                    (1,), mode=lax.GatherScatterMode.PROMISE_IN_BOUNDS)
                rows[rk, e, pl.ds(0, 16)] = rows[rk, e, pl.ds(0, 16)] * v
                rows[rk, e, pl.ds(16, 16)] = rows[rk, e, pl.ds(16, 16)] * v
        doff = lax.rem(blk, DBRING) * BLKE + lax.rem(k, BPB) * CHUNK
        pltpu.async_copy(rows.at[rk], accum.at[didx.at[pl.ds(doff, CHUNK)]],
                         sem_sc, add=True)
        return carry

    lax.fori_loop(0, NCHUNK, chunk_body, 0)
    for _ in range(SCD):
        drain_rows(sem_sc)
    plsc.subcore_barrier()
    pltpu.sync_copy(accum.at[pl.ds(s * STRIPE, STRIPE)],
                    partials.at[c, pl.ds(s * STRIPE, STRIPE)])


_propagate = functools.partial(
    pl.kernel,
    out_type=jax.ShapeDtypeStruct((NC, N_NODES, D), jnp.float32),
    mesh=_MESH,
    compiler_params=pltpu.CompilerParams(use_tc_tiling_on_sc=False, needs_layout_passes=False),
    scratch_types=[
        pltpu.VMEM_SHARED((N_NODES, D), jnp.float32),
        pltpu.VMEM((BRING * BLKE,), jnp.int32),
        pltpu.VMEM((DBRING * BLKE,), jnp.int32),
        pltpu.VMEM((BRING * BLKE + 16,), jnp.float32),
        pltpu.VMEM((RRING, CHUNK, D), jnp.float32),
        pltpu.SemaphoreType.DMA,
        pltpu.SemaphoreType.DMA,
        pltpu.SemaphoreType.DMA,
    ],
)(_propagate_body)


MROWS = 1563
MSUB = 521
U_RPT = N_USERS // NW
I_SUB_A = 469
I_SUB_B = 468


def _merge_body(p_r, out_r, a, b):
    c = lax.axis_index("c")
    s = lax.axis_index("s")
    wid = s * NC + c

    def seg(r0, n):
        pltpu.sync_copy(p_r.at[0, pl.ds(r0, n)], a.at[pl.ds(0, n)])
        pltpu.sync_copy(p_r.at[1, pl.ds(r0, n)], b.at[pl.ds(0, n)])

        def add(r, carry):
            a[r, pl.ds(0, 16)] = a[r, pl.ds(0, 16)] + b[r, pl.ds(0, 16)]
            a[r, pl.ds(16, 16)] = a[r, pl.ds(16, 16)] + b[r, pl.ds(16, 16)]
            return carry

        lax.fori_loop(0, n, add, 0)
        pltpu.sync_copy(a.at[pl.ds(0, n)], out_r.at[pl.ds(r0, n)])

    @pl.when(wid < NW - 1)
    def _():
        for t in range(3):
            seg(wid * MROWS + t * MSUB, MSUB)

    @pl.when(wid == NW - 1)
    def _():
        base = (NW - 1) * MROWS
        seg(base, MSUB)
        seg(base + MSUB, MSUB)
        seg(base + 2 * MSUB, N_NODES - base - 2 * MSUB)


_merge = functools.partial(
    pl.kernel,
    out_type=jax.ShapeDtypeStruct((N_NODES, D), jnp.float32),
    mesh=_MESH,
    compiler_params=pltpu.CompilerParams(use_tc_tiling_on_sc=False, needs_layout_passes=False),
    scratch_types=[
        pltpu.VMEM((MSUB, D), jnp.float32),
        pltpu.VMEM((MSUB, D), jnp.float32),
    ],
)(_merge_body)


def _final_body(p_r, e1_r, e2_r, ua_r, ia_r, ul_r, il_r, b1, b2, b3, b4):
    c = lax.axis_index("c")
    s = lax.axis_index("s")
    wid = s * NC + c
    third = jnp.float32(1.0 / 3.0)

    def seg(in_r0, out_r0, n, lay_out, all_out):
        pltpu.sync_copy(e1_r.at[pl.ds(in_r0, n)], b1.at[pl.ds(0, n)])
        pltpu.sync_copy(e2_r.at[pl.ds(in_r0, n)], b2.at[pl.ds(0, n)])
        pltpu.sync_copy(p_r.at[0, pl.ds(in_r0, n)], b3.at[pl.ds(0, n)])
        pltpu.sync_copy(p_r.at[1, pl.ds(in_r0, n)], b4.at[pl.ds(0, n)])

        def add(r, carry):
            b3[r, pl.ds(0, 16)] = b3[r, pl.ds(0, 16)] + b4[r, pl.ds(0, 16)]
            b3[r, pl.ds(16, 16)] = b3[r, pl.ds(16, 16)] + b4[r, pl.ds(16, 16)]
            return carry

        lax.fori_loop(0, n, add, 0)

        def mean(r, carry):
            for h in (0, 16):
                b4[r, pl.ds(h, 16)] = (b1[r, pl.ds(h, 16)]
                                       + b2[r, pl.ds(h, 16)]
                                       + b3[r, pl.ds(h, 16)]) * third
            return carry

        lax.fori_loop(0, n, mean, 0)
        pltpu.sync_copy(b1.at[pl.ds(0, n)], lay_out.at[pl.ds(out_r0, n), 0])
        pltpu.sync_copy(b2.at[pl.ds(0, n)], lay_out.at[pl.ds(out_r0, n), 1])
        pltpu.sync_copy(b3.at[pl.ds(0, n)], lay_out.at[pl.ds(out_r0, n), 2])
        pltpu.sync_copy(b4.at[pl.ds(0, n)], all_out.at[pl.ds(out_r0, n)])

    seg(wid * U_RPT, wid * U_RPT, U_RPT, ul_r, ua_r)

    @pl.when(wid < NS)
    def _():
        r0 = wid * (2 * I_SUB_A)
        seg(N_USERS + r0, r0, I_SUB_A, il_r, ia_r)
        seg(N_USERS + r0 + I_SUB_A, r0 + I_SUB_A, I_SUB_A, il_r, ia_r)

    @pl.when(wid >= NS)
    def _():
        r0 = NS * (2 * I_SUB_A) + (wid - NS) * (I_SUB_A + I_SUB_B)
        seg(N_USERS + r0, r0, I_SUB_A, il_r, ia_r)
        seg(N_USERS + r0 + I_SUB_A, r0 + I_SUB_A, I_SUB_B, il_r, ia_r)


_final = functools.partial(
    pl.kernel,
    out_type=(jax.ShapeDtypeStruct((N_USERS, D), jnp.float32),
              jax.ShapeDtypeStruct((N_ITEMS, D), jnp.float32),
              jax.ShapeDtypeStruct((N_USERS, N_LAY, D), jnp.float32),
              jax.ShapeDtypeStruct((N_ITEMS, N_LAY, D), jnp.float32)),
    mesh=_MESH,
    compiler_params=pltpu.CompilerParams(use_tc_tiling_on_sc=False, needs_layout_passes=False),
    scratch_types=[
        pltpu.VMEM((U_RPT, D), jnp.float32),
        pltpu.VMEM((U_RPT, D), jnp.float32),
        pltpu.VMEM((U_RPT, D), jnp.float32),
        pltpu.VMEM((U_RPT, D), jnp.float32),
    ],
)(_final_body)


def kernel(user_emb, item_emb, adj_vals, edge_index):
    ego = jnp.concatenate([user_emb, item_emb], axis=0)
    zeros = jnp.zeros((STRIPE, D), jnp.float32)

    p1 = _propagate(ego, edge_index, adj_vals, zeros)
    e1 = _merge(p1)
    p2 = _propagate(e1, edge_index, adj_vals, zeros)
    e2 = _merge(p2)
    p3 = _propagate(e2, edge_index, adj_vals, zeros)
    return _final(p3, e1, e2)
